# baseline (speedup 1.0000x reference)
import jax
import jax.numpy as jnp
from jax.experimental import pallas as pl
from jax.experimental.pallas import tpu as pltpu

_BM = 200


def _gcn_kernel(seq_ref, wt_ref, adj_ref, bias_ref, a_ref, o_ref, fts_ref):
    @pl.when(pl.program_id(0) == 0)
    def _():
        fts_ref[...] = jnp.dot(
            seq_ref[...].astype(jnp.bfloat16),
            wt_ref[...].astype(jnp.bfloat16),
            preferred_element_type=jnp.float32,
        ).astype(jnp.bfloat16)

    out = jnp.dot(
        adj_ref[...].astype(jnp.bfloat16),
        fts_ref[...],
        preferred_element_type=jnp.float32,
    ) + bias_ref[...]
    o_ref[...] = jnp.where(out > 0, out, a_ref[0, 0] * out)


def kernel(seq, adj, W, bias, prelu_a):
    n, d_in = seq.shape
    d_out = W.shape[0]

    out = pl.pallas_call(
        _gcn_kernel,
        grid=(n // _BM,),
        in_specs=[
            pl.BlockSpec((n, d_in), lambda i: (0, 0)),
            pl.BlockSpec((d_in, d_out), lambda i: (0, 0)),
            pl.BlockSpec((_BM, n), lambda i: (i, 0)),
            pl.BlockSpec((1, d_out), lambda i: (0, 0)),
            pl.BlockSpec((1, 1), lambda i: (0, 0)),
        ],
        out_specs=pl.BlockSpec((_BM, d_out), lambda i: (i, 0)),
        out_shape=jax.ShapeDtypeStruct((n, d_out), jnp.float32),
        scratch_shapes=[pltpu.VMEM((n, d_out), jnp.bfloat16)],
        compiler_params=pltpu.CompilerParams(
            dimension_semantics=("arbitrary",),
            vmem_limit_bytes=120 * 1024 * 1024,
        ),
    )(seq, W.T, adj, bias.reshape(1, d_out), prelu_a.reshape(1, 1))
    return out


# speedup vs baseline: 1.0169x; 1.0169x over previous
import jax
import jax.numpy as jnp
from jax.experimental import pallas as pl
from jax.experimental.pallas import tpu as pltpu

_BM = 400


def _gcn_kernel(seq_ref, wt_ref, adj_ref, bias_ref, a_ref, o_ref, fts_ref):
    @pl.when(pl.program_id(0) == 0)
    def _():
        fts_ref[...] = jnp.dot(
            seq_ref[...], wt_ref[...], preferred_element_type=jnp.float32
        )

    out = jnp.dot(
        adj_ref[...], fts_ref[...], preferred_element_type=jnp.float32
    ) + bias_ref[...]
    o_ref[...] = jnp.where(out > 0, out, a_ref[0, 0] * out)


def kernel(seq, adj, W, bias, prelu_a):
    n, d_in = seq.shape
    d_out = W.shape[0]

    out = pl.pallas_call(
        _gcn_kernel,
        grid=(n // _BM,),
        in_specs=[
            pl.BlockSpec((n, d_in), lambda i: (0, 0)),
            pl.BlockSpec((d_in, d_out), lambda i: (0, 0)),
            pl.BlockSpec((_BM, n), lambda i: (i, 0)),
            pl.BlockSpec((1, d_out), lambda i: (0, 0)),
            pl.BlockSpec((1, 1), lambda i: (0, 0)),
        ],
        out_specs=pl.BlockSpec((_BM, d_out), lambda i: (i, 0)),
        out_shape=jax.ShapeDtypeStruct((n, d_out), jnp.float32),
        scratch_shapes=[pltpu.VMEM((n, d_out), jnp.float32)],
        compiler_params=pltpu.CompilerParams(
            dimension_semantics=("arbitrary",),
            vmem_limit_bytes=62 * 1024 * 1024,
        ),
    )(seq, W.T, adj, bias.reshape(1, d_out), prelu_a.reshape(1, 1))
    return out


# speedup vs baseline: 1.0199x; 1.0030x over previous
import jax
import jax.numpy as jnp
from jax.experimental import pallas as pl
from jax.experimental.pallas import tpu as pltpu

_BM = 512


def _gcn_kernel(seq_ref, wt_ref, adj_ref, bias_ref, a_ref, o_ref, fts_ref):
    @pl.when(pl.program_id(0) == 0)
    def _():
        fts_ref[...] = jnp.dot(
            seq_ref[...], wt_ref[...], preferred_element_type=jnp.float32
        )

    out = jnp.dot(
        adj_ref[...], fts_ref[...], preferred_element_type=jnp.float32
    ) + bias_ref[...]
    o_ref[...] = jnp.where(out > 0, out, a_ref[0, 0] * out)


def kernel(seq, adj, W, bias, prelu_a):
    n, d_in = seq.shape
    d_out = W.shape[0]

    out = pl.pallas_call(
        _gcn_kernel,
        grid=((n + _BM - 1) // _BM,),
        in_specs=[
            pl.BlockSpec((n, d_in), lambda i: (0, 0)),
            pl.BlockSpec((d_in, d_out), lambda i: (0, 0)),
            pl.BlockSpec((_BM, n), lambda i: (i, 0)),
            pl.BlockSpec((1, d_out), lambda i: (0, 0)),
            pl.BlockSpec((1, 1), lambda i: (0, 0)),
        ],
        out_specs=pl.BlockSpec((_BM, d_out), lambda i: (i, 0)),
        out_shape=jax.ShapeDtypeStruct((n, d_out), jnp.float32),
        scratch_shapes=[pltpu.VMEM((n, d_out), jnp.float32)],
        compiler_params=pltpu.CompilerParams(
            dimension_semantics=("arbitrary",),
            vmem_limit_bytes=64 * 1024 * 1024,
        ),
    )(seq, W.T, adj, bias.reshape(1, d_out), prelu_a.reshape(1, 1))
    return out
